# Initial kernel scaffold; baseline (speedup 1.0000x reference)
import jax
import jax.numpy as jnp
from jax.experimental import pallas as pl


def kernel(x, edge_index, batch, W1, b1, W2, b2, Wh1, bh1, Wh2, bh2):
    raise NotImplementedError("write your pallas kernel here")


# speedup vs baseline: 8.0681x; 8.0681x over previous
import dataclasses
import functools

import jax
import jax.numpy as jnp
from jax import lax
from jax.experimental import pallas as pl
from jax.experimental.pallas import tpu as pltpu
from jax.experimental.pallas import tpu_sc as plsc

N = 10000
E = 160000
D_IN = 256
HID = 512
G = 128

NCH = 4
FC = 128
NPAD = 10112
EB = 128
NB = 79
EPAD = 16 * NB * EB
RPT = NPAD // 16
HR = 80

_f32 = jnp.float32
_i32 = jnp.int32


def _sc_compiler_params():
    cp = pltpu.CompilerParams()
    if "needs_layout_passes" in pltpu.CompilerParams.__dataclass_fields__:
        cp = dataclasses.replace(cp, needs_layout_passes=False)
    return cp


def _sc_degree(dst_flat):
    mesh = plsc.VectorSubcoreMesh(core_axis_name="c", subcore_axis_name="s")
    ept = EPAD // 16

    @functools.partial(
        pl.kernel,
        out_type=jax.ShapeDtypeStruct((HR, FC), _f32),
        mesh=mesh,
        compiler_params=_sc_compiler_params(),
        scratch_types=[
            pltpu.VMEM((ept,), _i32),
            pltpu.VMEM((HR, FC), _f32),
            pltpu.VMEM((1, HR), _i32),
            pltpu.VMEM_SHARED((HR, FC), _f32),
        ],
    )
    def k(dst_hbm, out_hbm, dst_v, hist_v, rows_v, hist_sh):
        core = lax.axis_index("c")
        sub = lax.axis_index("s")

        @pl.when(core == 0)
        def _():
            pltpu.sync_copy(dst_hbm.at[pl.ds(sub * ept, ept)], dst_v)

            @pl.loop(0, HR)
            def _(r):
                @pl.loop(0, FC, step=16)
                def _(cc):
                    hist_v[r, pl.ds(cc, 16)] = jnp.zeros((16,), _f32)

            @pl.loop(0, HR, step=16)
            def _(r):
                rows_v[0, pl.ds(r, 16)] = jnp.arange(16, dtype=_i32) + r

            ones = jnp.ones((16,), _f32)

            @pl.loop(0, ept, step=16)
            def _(i):
                dv = dst_v[pl.ds(i, 16)]
                ri = lax.shift_right_logical(dv, 7)
                ci = lax.bitwise_and(dv, 127)
                plsc.addupdate_scatter(hist_v, [ri, ci], ones)

            @pl.when(sub == 0)
            def _():
                pltpu.sync_copy(hist_v, hist_sh)
            plsc.subcore_barrier()

            @pl.when(sub != 0)
            def _():
                pltpu.sync_copy(hist_v, hist_sh.at[rows_v.at[0]], add=True)
            plsc.subcore_barrier()

            @pl.when(sub == 0)
            def _():
                pltpu.sync_copy(hist_sh, out_hbm)

    return k(dst_flat)


def _sc_aggregate(hs_flat, src4, dst3):
    mesh = plsc.VectorSubcoreMesh(core_axis_name="c", subcore_axis_name="s")

    @functools.partial(
        pl.kernel,
        out_type=jax.ShapeDtypeStruct((NCH * NPAD, FC), _f32),
        mesh=mesh,
        compiler_params=_sc_compiler_params(),
        scratch_types=[
            pltpu.VMEM((NB, EB), _i32),
            pltpu.VMEM((NB, EB), _i32),
            pltpu.VMEM((EB, FC), _f32),
            pltpu.VMEM_SHARED((NPAD, FC), _f32),
        ],
    )
    def k(hs_hbm, src_hbm, dst_hbm, out_hbm, src_v, dst_v, rows_v, agg_sh):
        core = lax.axis_index("c")
        sub = lax.axis_index("s")
        pltpu.sync_copy(dst_hbm.at[sub], dst_v)
        for cc in range(2):
            c = core * 2 + cc
            pltpu.sync_copy(
                hs_hbm.at[pl.ds(c * NPAD + sub * RPT, RPT)],
                agg_sh.at[pl.ds(sub * RPT, RPT)],
            )
            pltpu.sync_copy(src_hbm.at[c * 16 + sub], src_v)
            plsc.subcore_barrier()

            @pl.loop(0, NB)
            def _(j):
                pltpu.sync_copy(hs_hbm.at[src_v.at[j]], rows_v)
                pltpu.sync_copy(rows_v, agg_sh.at[dst_v.at[j]], add=True)

            plsc.subcore_barrier()
            pltpu.sync_copy(
                agg_sh.at[pl.ds(sub * RPT, RPT)],
                out_hbm.at[pl.ds(c * NPAD + sub * RPT, RPT)],
            )

    return k(hs_flat, src4, dst3)


_BM = 1000


def _tc_matmul1(x, W1, deg_col):
    def body(x_ref, w_ref, deg_ref, o_ref):
        acc = jnp.dot(x_ref[...], w_ref[...], preferred_element_type=_f32)
        acc = acc * lax.rsqrt(deg_ref[...] + 1.0)
        for c in range(NCH):
            o_ref[c] = acc[:, c * FC:(c + 1) * FC]

    return pl.pallas_call(
        body,
        grid=(N // _BM,),
        in_specs=[
            pl.BlockSpec((_BM, D_IN), lambda m: (m, 0)),
            pl.BlockSpec((D_IN, HID), lambda m: (0, 0)),
            pl.BlockSpec((_BM, 1), lambda m: (m, 0)),
        ],
        out_specs=pl.BlockSpec((NCH, _BM, FC), lambda m: (0, m, 0)),
        out_shape=jax.ShapeDtypeStruct((NCH, NPAD, FC), _f32),
    )(x, W1, deg_col)


def _tc_matmul2(agg1, W2r, b1r, deg_col):
    def body(a_ref, w_ref, b_ref, deg_ref, o_ref):
        dinv = lax.rsqrt(deg_ref[...] + 1.0)
        acc = jnp.zeros((_BM, HID), _f32)
        for c in range(NCH):
            h = jnp.maximum(a_ref[c] * dinv + b_ref[c], 0.0)
            acc += jnp.dot(h, w_ref[c], preferred_element_type=_f32)
        acc = acc * dinv
        for c in range(NCH):
            o_ref[c] = acc[:, c * FC:(c + 1) * FC]

    return pl.pallas_call(
        body,
        grid=(N // _BM,),
        in_specs=[
            pl.BlockSpec((NCH, _BM, FC), lambda m: (0, m, 0)),
            pl.BlockSpec((NCH, FC, HID), lambda m: (0, 0, 0)),
            pl.BlockSpec((NCH, 1, FC), lambda m: (0, 0, 0)),
            pl.BlockSpec((_BM, 1), lambda m: (m, 0)),
        ],
        out_specs=pl.BlockSpec((NCH, _BM, FC), lambda m: (0, m, 0)),
        out_shape=jax.ShapeDtypeStruct((NCH, NPAD, FC), _f32),
    )(agg1, W2r, b1r, deg_col)


def _tc_pool(agg2, b2r, deg_col, batch_row):
    def body(a_ref, b_ref, deg_ref, bt_ref, o_ref, cnt_ref):
        kk = pl.program_id(0)
        dinv = lax.rsqrt(deg_ref[...] + 1.0)
        gids = lax.broadcasted_iota(_i32, (G, _BM), 0)
        ind = (gids == bt_ref[0]).astype(_f32)

        @pl.when(kk == 0)
        def _():
            o_ref[...] = jnp.zeros_like(o_ref)
            cnt_ref[...] = jnp.zeros_like(cnt_ref)

        cnt_ref[...] += jnp.sum(ind, axis=1, keepdims=True)
        for c in range(NCH):
            h = jnp.maximum(a_ref[c] * dinv + b_ref[c], 0.0)
            o_ref[c] += jnp.dot(ind, h, preferred_element_type=_f32)

    return pl.pallas_call(
        body,
        grid=(N // _BM,),
        in_specs=[
            pl.BlockSpec((NCH, _BM, FC), lambda k: (0, k, 0)),
            pl.BlockSpec((NCH, 1, FC), lambda k: (0, 0, 0)),
            pl.BlockSpec((_BM, 1), lambda k: (k, 0)),
            pl.BlockSpec((1, 1, _BM), lambda k: (k, 0, 0)),
        ],
        out_specs=[
            pl.BlockSpec((NCH, G, FC), lambda k: (0, 0, 0)),
            pl.BlockSpec((G, 1), lambda k: (0, 0)),
        ],
        out_shape=[
            jax.ShapeDtypeStruct((NCH, G, FC), _f32),
            jax.ShapeDtypeStruct((G, 1), _f32),
        ],
    )(agg2, b2r, deg_col, batch_row)


def _tc_head(pooled_raw, cnt, Wh1r, bh1_row, Wh2, bh2_row):
    def body(p_ref, c_ref, w1_ref, b1_ref, w2_ref, b2_ref, o_ref):
        scale = 1.0 / jnp.maximum(c_ref[...], 1.0)
        z = jnp.zeros((G, HID), _f32)
        for c in range(NCH):
            z += jnp.dot(p_ref[c] * scale, w1_ref[c],
                         preferred_element_type=_f32)
        z = jnp.maximum(z + b1_ref[...], 0.0)
        o_ref[...] = jnp.dot(z, w2_ref[...],
                             preferred_element_type=_f32) + b2_ref[...]

    return pl.pallas_call(
        body,
        in_specs=[
            pl.BlockSpec((NCH, G, FC), lambda: (0, 0, 0)),
            pl.BlockSpec((G, 1), lambda: (0, 0)),
            pl.BlockSpec((NCH, FC, HID), lambda: (0, 0, 0)),
            pl.BlockSpec((1, HID), lambda: (0, 0)),
            pl.BlockSpec((HID, 1), lambda: (0, 0)),
            pl.BlockSpec((1, 1), lambda: (0, 0)),
        ],
        out_specs=pl.BlockSpec((G, 1), lambda: (0, 0)),
        out_shape=jax.ShapeDtypeStruct((G, 1), _f32),
    )(pooled_raw, cnt, Wh1r, bh1_row, Wh2, bh2_row)


def kernel(x, edge_index, batch, W1, b1, W2, b2, Wh1, bh1, Wh2, bh2):
    src = edge_index[0]
    dst = edge_index[1]
    pad = EPAD - E
    src_p = jnp.concatenate([src, jnp.zeros((pad,), _i32)])
    dst_p = jnp.concatenate([dst, jnp.full((pad,), N, _i32)])
    dst3 = dst_p.reshape(16, NB, EB)
    src4 = (src_p[None, :]
            + (jnp.arange(NCH, dtype=_i32) * NPAD)[:, None]).reshape(
                NCH * 16, NB, EB)

    deg_hist = _sc_degree(dst_p)
    deg_col = deg_hist.reshape(-1)[:N, None]

    hs1 = _tc_matmul1(x, W1, deg_col)
    agg1 = _sc_aggregate(hs1.reshape(NCH * NPAD, FC), src4, dst3)

    W2r = W2.reshape(NCH, FC, HID)
    b1r = b1.reshape(NCH, 1, FC)
    hs2 = _tc_matmul2(agg1.reshape(NCH, NPAD, FC), W2r, b1r, deg_col)
    agg2 = _sc_aggregate(hs2.reshape(NCH * NPAD, FC), src4, dst3)

    b2r = b2.reshape(NCH, 1, FC)
    batch_row = batch.reshape(N // _BM, 1, _BM)
    pooled_raw, cnt = _tc_pool(agg2.reshape(NCH, NPAD, FC), b2r, deg_col,
                               batch_row)

    out = _tc_head(pooled_raw, cnt, Wh1.reshape(NCH, FC, HID),
                   bh1.reshape(1, HID), Wh2, bh2.reshape(1, 1))
    return out.reshape(-1)
